# baseline (speedup 1.0000x reference)
import jax
import jax.numpy as jnp
from jax.experimental import pallas as pl


def _row_body(conf_ref, cat_ref, out_ref):
    x = conf_ref[0]
    m = jnp.max(x, axis=-1)
    s = jnp.sum(jnp.exp(x - m[:, None]), axis=-1)
    loss = m + jnp.log(s) - x[:, 0]
    loss = jnp.where(cat_ref[0, 0] > 0, -jnp.inf, loss)
    rmax = jnp.max(loss)
    t = (rmax * 0.0) * 0.0
    out_ref[0, 0] = loss + t


def kernel(predicted_boxes, confidences, target_boxes, target_categories):
    B, N, C = confidences.shape
    out = pl.pallas_call(
        _row_body,
        grid=(B,),
        in_specs=[
            pl.BlockSpec((1, N, C), lambda b: (b, 0, 0)),
            pl.BlockSpec((1, 1, N), lambda b: (b, 0, 0)),
        ],
        out_specs=pl.BlockSpec((1, 1, N), lambda b: (b, 0, 0)),
        out_shape=jax.ShapeDtypeStruct((B, 1, N), jnp.float32),
    )(confidences, target_categories.astype(jnp.int32).reshape(B, 1, N))
    return jax.lax.stop_gradient(out.reshape(B, N))
